# Initial kernel scaffold; baseline (speedup 1.0000x reference)
import jax
import jax.numpy as jnp
from jax.experimental import pallas as pl


def kernel(params, x_a, edge_attr_a, edge_index_a, edge_attr_b, edge_index_b, x_a_batch):
    raise NotImplementedError("write your pallas kernel here")


# speedup vs baseline: 1.0888x; 1.0888x over previous
import functools

import jax
import jax.numpy as jnp
from jax import lax
from jax.experimental import pallas as pl
from jax.experimental.pallas import tpu as pltpu
from jax.experimental.pallas import tpu_sc as plsc

_EMB = 64
_L = 8
_NG = 256
_LANES = 16
_K = 128
_G = 2000


def _zero_rows(buf, n):
    zeros16f = jnp.zeros((_LANES,), jnp.float32)

    def zrow(i, _):
        for c in range(_EMB // _LANES):
            buf[i, pl.ds(c * _LANES, _LANES)] = zeros16f
        return 0
    lax.fori_loop(0, n, zrow, 0)


def _make_mp_node(nE, D_pad, cpr):
    ch_rows = cpr * _LANES * _K
    assert ch_rows == D_pad
    EPT = nE // 2 // _LANES
    assert EPT % _K == 0
    stripe = ch_rows // _LANES
    mesh = plsc.VectorSubcoreMesh(core_axis_name="c", subcore_axis_name="s")

    @functools.partial(
        pl.kernel,
        out_type=jax.ShapeDtypeStruct((2 * D_pad, _EMB), jnp.float32),
        mesh=mesh,
        compiler_params=pltpu.CompilerParams(use_tc_tiling_on_sc=False),
        scratch_types=[
            pltpu.VMEM((_K,), jnp.int32),
            pltpu.VMEM((_K, _EMB), jnp.float32),
            pltpu.VMEM((_K, _EMB), jnp.float32),
            pltpu.VMEM_SHARED((ch_rows, _EMB), jnp.float32),
            pltpu.SemaphoreType.DMA,
        ],
    )
    def mp(x_hbm, e_hbm, src_hbm, dst_hbm, out_hbm,
           idx, xrows, erows, agg, sem1):
        cid = lax.axis_index("c")
        sid = lax.axis_index("s")
        tbase = (cid * _LANES + sid) * EPT

        _zero_rows(xrows, _K)
        for r in range(cpr):
            pltpu.sync_copy(xrows, agg.at[pl.ds(sid * stripe + r * _K, _K)])
        plsc.subcore_barrier()

        def blk(b, _):
            base = tbase + b * _K
            pltpu.sync_copy(src_hbm.at[pl.ds(base, _K)], idx)
            c1 = pltpu.async_copy(x_hbm.at[idx], xrows, sem1)
            pltpu.sync_copy(e_hbm.at[pl.ds(base, _K)], erows)
            c1.wait()

            def row(i, _):
                for c in range(_EMB // _LANES):
                    sl = pl.ds(c * _LANES, _LANES)
                    xrows[i, sl] = jnp.maximum(xrows[i, sl] + erows[i, sl], 0.0)
                return 0
            lax.fori_loop(0, _K, row, 0)
            pltpu.sync_copy(dst_hbm.at[pl.ds(base, _K)], idx)
            pltpu.sync_copy(xrows, agg.at[idx], add=True)
            return 0
        lax.fori_loop(0, EPT // _K, blk, 0)
        plsc.subcore_barrier()

        for r in range(cpr):
            rbase = sid * stripe + r * _K
            pltpu.sync_copy(agg.at[pl.ds(rbase, _K)],
                            out_hbm.at[pl.ds(cid * ch_rows + rbase, _K)])

    return mp


def _make_mp_edge(nE, n_chunks, cpr):
    ch_rows = cpr * _LANES * _K
    D_pad = n_chunks * ch_rows
    ch_per_sc = n_chunks // 2
    EPT = nE // _LANES
    assert EPT % _K == 0
    stripe = ch_rows // _LANES
    mesh = plsc.VectorSubcoreMesh(core_axis_name="c", subcore_axis_name="s")

    @functools.partial(
        pl.kernel,
        out_type=jax.ShapeDtypeStruct((D_pad, _EMB), jnp.float32),
        mesh=mesh,
        compiler_params=pltpu.CompilerParams(use_tc_tiling_on_sc=False),
        scratch_types=[
            pltpu.VMEM((_K,), jnp.int32),
            pltpu.VMEM((_K,), jnp.int32),
            pltpu.VMEM((_K, _EMB), jnp.float32),
            pltpu.VMEM((_K, _EMB), jnp.float32),
            pltpu.VMEM_SHARED((ch_rows + 8, _EMB), jnp.float32),
            pltpu.SemaphoreType.DMA,
        ],
    )
    def mp(x_hbm, e_hbm, src_hbm, dst_hbm, out_hbm,
           idx, dloc, xrows, erows, agg, sem1):
        cid = lax.axis_index("c")
        sid = lax.axis_index("s")
        tbase = sid * EPT

        for ch in range(ch_per_sc):
            lo = (cid * ch_per_sc + ch) * ch_rows

            _zero_rows(xrows, _K)
            for r in range(cpr):
                pltpu.sync_copy(xrows, agg.at[pl.ds(sid * stripe + r * _K, _K)])

            @pl.when(sid == 0)
            def _():
                pltpu.sync_copy(xrows.at[pl.ds(0, 8)], agg.at[pl.ds(ch_rows, 8)])
            plsc.subcore_barrier()

            def blk(b, _):
                base = tbase + b * _K
                pltpu.sync_copy(src_hbm.at[pl.ds(base, _K)], idx)
                c1 = pltpu.async_copy(x_hbm.at[idx], xrows, sem1)
                pltpu.sync_copy(e_hbm.at[pl.ds(base, _K)], erows)
                pltpu.sync_copy(dst_hbm.at[pl.ds(base, _K)], idx)
                c1.wait()

                def row(i, _):
                    for c in range(_EMB // _LANES):
                        sl = pl.ds(c * _LANES, _LANES)
                        xrows[i, sl] = jnp.maximum(xrows[i, sl] + erows[i, sl], 0.0)
                    return 0
                lax.fori_loop(0, _K, row, 0)

                def redir(j, _):
                    dv = idx[pl.ds(j * _LANES, _LANES)]
                    m = (dv >= lo) & (dv < lo + ch_rows)
                    dloc[pl.ds(j * _LANES, _LANES)] = jnp.where(m, dv - lo, ch_rows)
                    return 0
                lax.fori_loop(0, _K // _LANES, redir, 0)
                pltpu.sync_copy(xrows, agg.at[dloc], add=True)
                return 0
            lax.fori_loop(0, EPT // _K, blk, 0)
            plsc.subcore_barrier()

            for r in range(cpr):
                rbase = sid * stripe + r * _K
                pltpu.sync_copy(agg.at[pl.ds(rbase, _K)],
                                out_hbm.at[pl.ds(lo + rbase, _K)])
            plsc.subcore_barrier()

    return mp


def _embed_nodes(x_a, emb):
    n, nf = x_a.shape
    BR = 2000
    nv = emb.shape[1]

    def body(x_ref, emb_ref, o_ref):
        ids = x_ref[...]
        iot = lax.broadcasted_iota(jnp.int32, (BR, nv), 1)
        acc = jnp.zeros((BR, _EMB), jnp.float32)
        for f in range(nf):
            oh = (iot == ids[:, f][:, None]).astype(jnp.float32)
            acc = acc + jnp.dot(oh, emb_ref[f], preferred_element_type=jnp.float32)
        o_ref[...] = acc

    return pl.pallas_call(
        body,
        grid=(n // BR,),
        in_specs=[pl.BlockSpec((BR, nf), lambda i: (i, 0)),
                  pl.BlockSpec(emb.shape, lambda i: (0, 0, 0))],
        out_specs=pl.BlockSpec((BR, _EMB), lambda i: (i, 0)),
        out_shape=jax.ShapeDtypeStruct((n, _EMB), jnp.float32),
    )(x_a, emb)


def _rbf_feats(x, ncent):
    n = x.shape[0]
    BR = 4096

    def body(x_ref, o_ref):
        cent = lax.broadcasted_iota(jnp.int32, (1, ncent), 1).astype(jnp.float32) * 0.1
        d = x_ref[...] - cent
        o_ref[...] = jnp.exp(-10.0 * d * d)

    return pl.pallas_call(
        body,
        grid=(n // BR,),
        in_specs=[pl.BlockSpec((BR, 1), lambda i: (i, 0))],
        out_specs=pl.BlockSpec((BR, ncent), lambda i: (i, 0)),
        out_shape=jax.ShapeDtypeStruct((n, ncent), jnp.float32),
    )(x)


def _edge_dense(ints, feats, W, b):
    n = ints.shape[0]
    BR = 4096
    kf = feats.shape[1]

    def body(i_ref, f_ref, w_ref, b_ref, o_ref):
        ids = i_ref[...].astype(jnp.int32)
        iot = lax.broadcasted_iota(jnp.int32, (BR, 16), 1)
        parts = [(iot == ids[:, f][:, None]).astype(jnp.float32) for f in range(3)]
        x = jnp.concatenate(parts + [f_ref[...]], axis=1)
        o_ref[...] = jnp.dot(x, w_ref[...], preferred_element_type=jnp.float32) + b_ref[...]

    return pl.pallas_call(
        body,
        grid=(n // BR,),
        in_specs=[pl.BlockSpec((BR, 3), lambda i: (i, 0)),
                  pl.BlockSpec((BR, kf), lambda i: (i, 0)),
                  pl.BlockSpec(W.shape, lambda i: (0, 0)),
                  pl.BlockSpec((1, _EMB), lambda i: (0, 0))],
        out_specs=pl.BlockSpec((BR, _EMB), lambda i: (i, 0)),
        out_shape=jax.ShapeDtypeStruct((n, _EMB), jnp.float32),
    )(ints, feats, W, b)


def _linbias(x, W, b):
    n, k = x.shape
    BR = 4096

    def body(x_ref, w_ref, b_ref, o_ref):
        o_ref[...] = jnp.dot(x_ref[...], w_ref[...],
                             preferred_element_type=jnp.float32) + b_ref[...]

    return pl.pallas_call(
        body,
        grid=(n // BR,),
        in_specs=[pl.BlockSpec((BR, k), lambda i: (i, 0)),
                  pl.BlockSpec(W.shape, lambda i: (0, 0)),
                  pl.BlockSpec((1, _EMB), lambda i: (0, 0))],
        out_specs=pl.BlockSpec((BR, _EMB), lambda i: (i, 0)),
        out_shape=jax.ShapeDtypeStruct((n, _EMB), jnp.float32),
    )(x, W, b)


def _mlp_block(agg, res, W1, b1, W2, b2, g, bb, scale, BR):
    n = res.shape[0]

    def body(a_ref, r_ref, w1, b1r, w2, b2r, gr, br, o_ref):
        h = jnp.dot(a_ref[...], w1[...], preferred_element_type=jnp.float32) + b1r[...]
        h = jnp.maximum(h, 0.0)
        h = jnp.dot(h, w2[...], preferred_element_type=jnp.float32) + b2r[...]
        m = jnp.mean(h, axis=-1, keepdims=True)
        d = h - m
        v = jnp.mean(d * d, axis=-1, keepdims=True)
        h = d * lax.rsqrt(v + 1e-5) * gr[...] + br[...]
        o_ref[...] = h * scale + r_ref[...]

    vec = pl.BlockSpec((1, _EMB), lambda i: (0, 0))
    return pl.pallas_call(
        body,
        grid=(n // BR,),
        in_specs=[pl.BlockSpec((BR, _EMB), lambda i: (i, 0)),
                  pl.BlockSpec((BR, _EMB), lambda i: (i, 0)),
                  pl.BlockSpec((_EMB, _EMB), lambda i: (0, 0)), vec,
                  pl.BlockSpec((_EMB, _EMB), lambda i: (0, 0)), vec, vec, vec],
        out_specs=pl.BlockSpec((BR, _EMB), lambda i: (i, 0)),
        out_shape=jax.ShapeDtypeStruct((n, _EMB), jnp.float32),
    )(agg, res, W1, b1, W2, b2, g, bb)


def _readout(x, batch3):
    steps, _, BR = batch3.shape

    def body(b_ref, x_ref, o_ref, sacc, scnt):
        i = pl.program_id(0)

        @pl.when(i == 0)
        def _():
            sacc[...] = jnp.zeros_like(sacc)
            scnt[...] = jnp.zeros_like(scnt)

        bid = b_ref[0, 0, :]
        oh = (lax.broadcasted_iota(jnp.int32, (_NG, BR), 0) == bid[None, :])
        oh = oh.astype(jnp.float32)
        sacc[...] += jnp.dot(oh, x_ref[...], preferred_element_type=jnp.float32)
        scnt[...] += jnp.sum(oh, axis=1, keepdims=True)

        @pl.when(i == steps - 1)
        def _():
            o_ref[...] = sacc[...] / jnp.maximum(scnt[...], 1.0)

    return pl.pallas_call(
        body,
        grid=(steps,),
        in_specs=[pl.BlockSpec((1, 1, BR), lambda i: (i, 0, 0)),
                  pl.BlockSpec((BR, _EMB), lambda i: (i, 0))],
        out_specs=pl.BlockSpec((_NG, _EMB), lambda i: (0, 0)),
        out_shape=jax.ShapeDtypeStruct((_NG, _EMB), jnp.float32),
        scratch_shapes=[pltpu.VMEM((_NG, _EMB), jnp.float32),
                        pltpu.VMEM((_NG, _EMB), jnp.float32)],
    )(batch3, x)


def _mlp_block2(agg, res, W1, b1, W2, b2, g, bb, scale, BR):
    n = res.shape[0]

    def body(a1_ref, a2_ref, r_ref, w1, b1r, w2, b2r, gr, br, o_ref):
        a = a1_ref[0] + a2_ref[0]
        h = jnp.dot(a, w1[...], preferred_element_type=jnp.float32) + b1r[...]
        h = jnp.maximum(h, 0.0)
        h = jnp.dot(h, w2[...], preferred_element_type=jnp.float32) + b2r[...]
        m = jnp.mean(h, axis=-1, keepdims=True)
        d = h - m
        v = jnp.mean(d * d, axis=-1, keepdims=True)
        h = d * lax.rsqrt(v + 1e-5) * gr[...] + br[...]
        o_ref[...] = h * scale + r_ref[...]

    vec = pl.BlockSpec((1, _EMB), lambda i: (0, 0))
    return pl.pallas_call(
        body,
        grid=(n // BR,),
        in_specs=[pl.BlockSpec((1, BR, _EMB), lambda i: (0, i, 0)),
                  pl.BlockSpec((1, BR, _EMB), lambda i: (1, i, 0)),
                  pl.BlockSpec((BR, _EMB), lambda i: (i, 0)),
                  pl.BlockSpec((_EMB, _EMB), lambda i: (0, 0)), vec,
                  pl.BlockSpec((_EMB, _EMB), lambda i: (0, 0)), vec, vec, vec],
        out_specs=pl.BlockSpec((BR, _EMB), lambda i: (i, 0)),
        out_shape=jax.ShapeDtypeStruct((n, _EMB), jnp.float32),
    )(agg, agg, res, W1, b1, W2, b2, g, bb)


def kernel(params, x_a, edge_attr_a, edge_index_a, edge_attr_b, edge_index_b, x_a_batch):
    p = params
    N = x_a.shape[0]
    EA = edge_attr_a.shape[0]
    EB = edge_attr_b.shape[0]
    EA_pad = 163840
    EB_pad = 327680
    ND_pad = 10240

    mp_node = _make_mp_node(EA_pad, ND_pad, cpr=5)
    mp_edge = _make_mp_edge(EB_pad, n_chunks=8, cpr=10)

    x_a32 = x_a.astype(jnp.int32)
    node_h = _embed_nodes(x_a32, p['atom_emb'])

    ea_int = jnp.concatenate(
        [edge_attr_a[:, :3], jnp.zeros((EA_pad - EA, 3), jnp.float32)])
    ea_f = jnp.concatenate(
        [edge_attr_a[:, 3:4], jnp.zeros((EA_pad - EA, 1), jnp.float32)])
    eb_f = jnp.concatenate(
        [edge_attr_b, jnp.zeros((EB_pad - EB, 1), jnp.float32)])
    feats_a = _rbf_feats(ea_f, 20)
    feats_b = _rbf_feats(eb_f, 32)

    W0 = jnp.concatenate([p['init_bond_emb'].reshape(48, _EMB), p['init_rbf_W']], axis=0)
    edge_h = _edge_dense(ea_int, feats_a, W0, p['init_rbf_b'][None, :])

    Wc = jnp.concatenate([p['bond_embs'].reshape(_L, 48, _EMB), p['rbf_W']], axis=1)

    i32 = jnp.int32
    src_a = jnp.concatenate([edge_index_a[0].astype(i32), jnp.zeros((EA_pad - EA,), i32)])
    dst_a = jnp.concatenate([edge_index_a[1].astype(i32),
                             jnp.full((EA_pad - EA,), N + 100, i32)])
    src_b = jnp.concatenate([edge_index_b[0].astype(i32), jnp.zeros((EB_pad - EB,), i32)])
    dst_b = jnp.concatenate([edge_index_b[1].astype(i32),
                             jnp.full((EB_pad - EB,), EA + 1000, i32)])

    ab, ba = p['ab'], p['ba']
    sn = float(N) ** -0.5
    se = float(EA) ** -0.5

    for l in range(_L):
        agg_n = mp_node(node_h, edge_h, src_a, dst_a).reshape(2, ND_pad, _EMB)
        node_new = _mlp_block2(agg_n, node_h,
                               ab['W1'][l], ab['b1'][l][None, :],
                               ab['W2'][l], ab['b2'][l][None, :],
                               ab['ln_g'][l][None, :], ab['ln_b'][l][None, :],
                               sn, BR=2000)
        cur = _edge_dense(ea_int, feats_a, Wc[l], p['rbf_b'][l][None, :])
        ang = _linbias(feats_b, p['ang_W'][l], p['ang_b'][l][None, :])
        agg_e = mp_edge(cur, ang, src_b, dst_b)
        edge_h = _mlp_block(agg_e, cur,
                            ba['W1'][l], ba['b1'][l][None, :],
                            ba['W2'][l], ba['b2'][l][None, :],
                            ba['ln_g'][l][None, :], ba['ln_b'][l][None, :],
                            se, BR=4096)
        node_h = node_new

    batch3 = x_a_batch.astype(jnp.int32).reshape(5, 1, 2000)
    return _readout(node_h, batch3)
